# Initial kernel scaffold; baseline (speedup 1.0000x reference)
import jax
import jax.numpy as jnp
from jax.experimental import pallas as pl


def kernel(x, edge_index, edge_weights, W1_rel, b1_rel, W1_root, W2_rel, b2_rel, W2_root):
    raise NotImplementedError("write your pallas kernel here")


# speedup vs baseline: 3.6360x; 3.6360x over previous
import dataclasses
import functools

import jax
import jax.numpy as jnp
from jax import lax
from jax.experimental import pallas as pl
from jax.experimental.pallas import tpu as pltpu
from jax.experimental.pallas import tpu_sc as plsc

N_NODES = 10000
N_EDGES = 320000
D = 128

NC = 2
NS = 16
CHUNK = 128
NCHUNK = 79
EDGES_PER_TILE = CHUNK * NCHUNK
E_PAD = NC * NS * EDGES_PER_TILE
ROWS_PER_TILE = 624
ROWS_TAIL = N_NODES - NS * ROWS_PER_TILE

_FB = D // 16


def _sc_scatter_body(hr_hbm, src_hbm, dst_hbm, w_hbm, zeros_hbm, out_hbm,
                     src_v, dst_v, w_v, rows_v, acc_sh, sem):
    cid = lax.axis_index("c")
    sid = lax.axis_index("s")

    pltpu.sync_copy(zeros_hbm.at[pl.ds(sid * ROWS_PER_TILE, ROWS_PER_TILE)],
                    acc_sh.at[pl.ds(sid * ROWS_PER_TILE, ROWS_PER_TILE)])

    @pl.when(sid == NS - 1)
    def _zero_tail():
        pltpu.sync_copy(zeros_hbm.at[pl.ds(NS * ROWS_PER_TILE, ROWS_TAIL)],
                        acc_sh.at[pl.ds(NS * ROWS_PER_TILE, ROWS_TAIL)])

    plsc.subcore_barrier()

    pltpu.sync_copy(src_hbm.at[cid, sid], src_v)
    pltpu.sync_copy(dst_hbm.at[cid, sid], dst_v)
    pltpu.sync_copy(w_hbm.at[cid, sid], w_v)

    @pl.loop(0, NCHUNK)
    def _chunk(j):
        pltpu.async_copy(hr_hbm.at[src_v.at[j]], rows_v, sem).wait()
        jv = jnp.full((16,), j, dtype=jnp.int32)

        @pl.loop(0, CHUNK)
        def _edge(e):
            wv = plsc.load_gather(w_v, [jv, jnp.full((16,), e, jnp.int32)])
            for fb in range(_FB):
                sl = pl.ds(fb * 16, 16)
                rows_v[e, sl] = rows_v[e, sl] * wv

        pltpu.sync_copy(rows_v, acc_sh.at[dst_v.at[j]], add=True)

    plsc.subcore_barrier()
    pltpu.sync_copy(acc_sh.at[pl.ds(sid * ROWS_PER_TILE, ROWS_PER_TILE)],
                    out_hbm.at[cid, pl.ds(sid * ROWS_PER_TILE, ROWS_PER_TILE)])

    @pl.when(sid == NS - 1)
    def _write_tail():
        pltpu.sync_copy(acc_sh.at[pl.ds(NS * ROWS_PER_TILE, ROWS_TAIL)],
                        out_hbm.at[cid, pl.ds(NS * ROWS_PER_TILE, ROWS_TAIL)])


def _sc_scatter(hr, src_p, dst_p, w_p, zeros):
    mesh = plsc.VectorSubcoreMesh(core_axis_name="c", subcore_axis_name="s")
    cp = pltpu.CompilerParams()
    if "needs_layout_passes" in pltpu.CompilerParams.__dataclass_fields__:
        cp = dataclasses.replace(cp, needs_layout_passes=False)
    kern = pl.kernel(
        _sc_scatter_body,
        compiler_params=cp,
        out_type=jax.ShapeDtypeStruct((NC, N_NODES, D), jnp.float32),
        mesh=mesh,
        scratch_types=[
            pltpu.VMEM((NCHUNK, CHUNK), jnp.int32),
            pltpu.VMEM((NCHUNK, CHUNK), jnp.int32),
            pltpu.VMEM((NCHUNK, CHUNK), jnp.float32),
            pltpu.VMEM((CHUNK, D), jnp.float32),
            pltpu.VMEM_SHARED((N_NODES, D), jnp.float32),
            pltpu.SemaphoreType.DMA,
        ],
    )
    return kern(hr, src_p, dst_p, w_p, zeros)


_BLK = 2000


def _pre_body(h_ref, wr_ref, wo_ref, b_ref, hr_ref, base_ref):
    h = h_ref[...]
    hr_ref[...] = jnp.dot(h, wr_ref[...], preferred_element_type=jnp.float32)
    base_ref[...] = (
        jnp.dot(h, wo_ref[...], preferred_element_type=jnp.float32)
        + b_ref[...]
    )


def _dense_pre(h, w_rel, w_root, b_rel):
    return pl.pallas_call(
        _pre_body,
        grid=(N_NODES // _BLK,),
        in_specs=[
            pl.BlockSpec((_BLK, D), lambda i: (i, 0)),
            pl.BlockSpec((D, D), lambda i: (0, 0)),
            pl.BlockSpec((D, D), lambda i: (0, 0)),
            pl.BlockSpec((1, D), lambda i: (0, 0)),
        ],
        out_specs=[
            pl.BlockSpec((_BLK, D), lambda i: (i, 0)),
            pl.BlockSpec((_BLK, D), lambda i: (i, 0)),
        ],
        out_shape=[
            jax.ShapeDtypeStruct((N_NODES, D), jnp.float32),
            jax.ShapeDtypeStruct((N_NODES, D), jnp.float32),
        ],
    )(h, w_rel, w_root, b_rel.reshape(1, D))


def _mid_body(p_ref, base_ref, wr_ref, wo_ref, b_ref, hr_ref, base2_ref):
    h = p_ref[0] + p_ref[1] + base_ref[...]
    h = jnp.where(h > 0, h, jnp.exp(jnp.minimum(h, 0.0)) - 1.0)
    hr_ref[...] = jnp.dot(h, wr_ref[...], preferred_element_type=jnp.float32)
    base2_ref[...] = (
        jnp.dot(h, wo_ref[...], preferred_element_type=jnp.float32)
        + b_ref[...]
    )


def _dense_mid(p, base, w_rel, w_root, b_rel):
    return pl.pallas_call(
        _mid_body,
        grid=(N_NODES // _BLK,),
        in_specs=[
            pl.BlockSpec((NC, _BLK, D), lambda i: (0, i, 0)),
            pl.BlockSpec((_BLK, D), lambda i: (i, 0)),
            pl.BlockSpec((D, D), lambda i: (0, 0)),
            pl.BlockSpec((D, D), lambda i: (0, 0)),
            pl.BlockSpec((1, D), lambda i: (0, 0)),
        ],
        out_specs=[
            pl.BlockSpec((_BLK, D), lambda i: (i, 0)),
            pl.BlockSpec((_BLK, D), lambda i: (i, 0)),
        ],
        out_shape=[
            jax.ShapeDtypeStruct((N_NODES, D), jnp.float32),
            jax.ShapeDtypeStruct((N_NODES, D), jnp.float32),
        ],
    )(p, base, w_rel, w_root, b_rel.reshape(1, D))


def _final_body(p_ref, base_ref, out_ref):
    out_ref[...] = p_ref[0] + p_ref[1] + base_ref[...]


def _dense_final(p, base):
    return pl.pallas_call(
        _final_body,
        grid=(N_NODES // _BLK,),
        in_specs=[
            pl.BlockSpec((NC, _BLK, D), lambda i: (0, i, 0)),
            pl.BlockSpec((_BLK, D), lambda i: (i, 0)),
        ],
        out_specs=pl.BlockSpec((_BLK, D), lambda i: (i, 0)),
        out_shape=jax.ShapeDtypeStruct((N_NODES, D), jnp.float32),
    )(p, base)


def kernel(x, edge_index, edge_weights, W1_rel, b1_rel, W1_root,
           W2_rel, b2_rel, W2_root):
    ei = edge_index.astype(jnp.int32)
    pad = E_PAD - N_EDGES
    src_p = jnp.pad(ei[0], (0, pad)).reshape(NC, NS, NCHUNK, CHUNK)
    dst_p = jnp.pad(ei[1], (0, pad)).reshape(NC, NS, NCHUNK, CHUNK)
    w_p = jnp.pad(edge_weights.astype(jnp.float32), (0, pad)).reshape(
        NC, NS, NCHUNK, CHUNK)
    zeros = jnp.zeros((N_NODES, D), jnp.float32)

    hr1, base1 = _dense_pre(x, W1_rel, W1_root, b1_rel)
    part1 = _sc_scatter(hr1, src_p, dst_p, w_p, zeros)
    hr2, base2 = _dense_mid(part1, base1, W2_rel, W2_root, b2_rel)
    part2 = _sc_scatter(hr2, src_p, dst_p, w_p, zeros)
    return _dense_final(part2, base2)
